# baseline (speedup 1.0000x reference)
import jax
import jax.numpy as jnp
import numpy as np
from jax.experimental import pallas as pl
from jax.experimental.pallas import tpu as pltpu

B, N, F = 8, 128, 256
H, C = 3, 32
HC = H * C
K = int(np.ceil(0.6 * N))
PAIR = 4


def _one_graph(xlT, xrT, ab, att3, we, cb, pw):
    e3 = (xlT.T[:, :, None] + xrT[None, :, :]
          + ab[:, None, :] * we[None, :, None])
    e3 = jnp.maximum(e3, 0.2 * e3)
    att3b = jnp.broadcast_to(att3[None], (N, H, HC))
    logits = jax.lax.dot_general(att3b, e3, (((2,), (1,)), ((0,), (0,))),
                                 preferred_element_type=jnp.float32)

    mask3 = (ab != 0.0)[:, None, :]
    logits = jnp.where(mask3, logits, -1e9)
    m = jnp.max(logits, axis=0, keepdims=True)
    p = jnp.exp(logits - m)
    den = jnp.sum(p, axis=0)

    outall = jax.lax.dot_general(xlT, p, (((1,), (0,)), ((), ())),
                                 preferred_element_type=jnp.float32)
    h_rows = [outall[h * C:(h + 1) * C, h, :] / den[h][None, :] for h in range(H)]
    hfullT = jnp.concatenate(h_rows, axis=0) + cb[:, None]

    inv_norm = jax.lax.rsqrt(jnp.sum(pw * pw))
    s = jnp.tanh(jax.lax.dot_general(pw[None, :], hfullT,
                                     (((1,), (0,)), ((), ())),
                                     preferred_element_type=jnp.float32) * inv_norm)
    sv = s[0]
    gt = (sv[:, None] > sv[None, :]).astype(jnp.float32)
    idx = jax.lax.broadcasted_iota(jnp.int32, (N, N), 0)
    idy = jax.lax.broadcasted_iota(jnp.int32, (N, N), 1)
    eq = ((sv[:, None] == sv[None, :]) & (idx < idy)).astype(jnp.float32)
    rank = jnp.sum(gt + eq, axis=0)
    w = jnp.where(rank < float(K), sv, 0.0)
    return jax.lax.dot_general(
        hfullT, w[:, None], (((1,), (0,)), ((), ())),
        preferred_element_type=jnp.float32)[:, 0] * (1.0 / K)


def _gat_topk_kernel(x_ref, adj_ref, wl_ref, bl_ref, wr_ref, br_ref,
                     att3_ref, we_ref, cb_ref, pw_ref, out_ref):
    att3 = att3_ref[...]
    bl, br, we, cb, pw = (bl_ref[0], br_ref[0], we_ref[0], cb_ref[0],
                          pw_ref[0])
    x2 = x_ref[...].reshape(PAIR * N, F)
    xlT_all = jax.lax.dot_general(wl_ref[...], x2, (((0,), (1,)), ((), ())),
                                  preferred_element_type=jnp.float32) + bl[:, None]
    xrT_all = jax.lax.dot_general(wr_ref[...], x2, (((0,), (1,)), ((), ())),
                                  preferred_element_type=jnp.float32) + br[:, None]
    for g in range(PAIR):
        out_ref[g, 0, :] = _one_graph(xlT_all[:, g * N:(g + 1) * N],
                                      xrT_all[:, g * N:(g + 1) * N],
                                      adj_ref[g], att3, we, cb, pw)


@jax.jit
def kernel(x, adj, Wl, bl, Wr, br, att, We, conv_bias, pool_w):
    att3 = (jnp.eye(H, dtype=jnp.float32)[:, :, None] * att[None, :, :]).reshape(H, HC)
    out = pl.pallas_call(
        _gat_topk_kernel,
        grid=(B // PAIR,),
        in_specs=[
            pl.BlockSpec((PAIR, N, F), lambda b: (b, 0, 0)),
            pl.BlockSpec((PAIR, N, N), lambda b: (b, 0, 0)),
            pl.BlockSpec((F, HC), lambda b: (0, 0)),
            pl.BlockSpec((1, HC), lambda b: (0, 0)),
            pl.BlockSpec((F, HC), lambda b: (0, 0)),
            pl.BlockSpec((1, HC), lambda b: (0, 0)),
            pl.BlockSpec((H, HC), lambda b: (0, 0)),
            pl.BlockSpec((1, HC), lambda b: (0, 0)),
            pl.BlockSpec((1, HC), lambda b: (0, 0)),
            pl.BlockSpec((1, HC), lambda b: (0, 0)),
        ],
        out_specs=pl.BlockSpec((PAIR, 1, HC), lambda b: (b, 0, 0)),
        out_shape=jax.ShapeDtypeStruct((B, 1, HC), jnp.float32),
        compiler_params=pltpu.CompilerParams(
            dimension_semantics=("arbitrary",)),
    )(x, adj, Wl, bl.reshape(1, HC), Wr, br.reshape(1, HC), att3,
      We.reshape(1, HC), conv_bias.reshape(1, HC), pool_w.reshape(1, HC))
    return out[:, 0, :]


# speedup vs baseline: 1.0231x; 1.0231x over previous
import jax
import jax.numpy as jnp
import numpy as np
from jax.experimental import pallas as pl
from jax.experimental.pallas import tpu as pltpu

B, N, F = 8, 128, 256
H, C = 3, 32
HC = H * C
K = int(np.ceil(0.6 * N))
PAIR = 4


def _one_graph(xb, ab, wl, bl, wr, br, att3, we, cb, pw):
    xlT = jax.lax.dot_general(wl, xb, (((0,), (1,)), ((), ())),
                              preferred_element_type=jnp.float32) + bl[:, None]
    xrT = jax.lax.dot_general(wr, xb, (((0,), (1,)), ((), ())),
                              preferred_element_type=jnp.float32) + br[:, None]

    e3 = (xlT.T[:, :, None] + xrT[None, :, :]
          + ab[:, None, :] * we[None, :, None])
    e3 = jnp.maximum(e3, 0.2 * e3)
    att3b = jnp.broadcast_to(att3[None], (N, H, HC))
    logits = jax.lax.dot_general(att3b, e3, (((2,), (1,)), ((0,), (0,))),
                                 preferred_element_type=jnp.float32)

    mask3 = (ab != 0.0)[:, None, :]
    logits = jnp.where(mask3, logits, -1e9)
    m = jnp.max(logits, axis=0, keepdims=True)
    p = jnp.exp(logits - m)
    den = jnp.sum(p, axis=0)

    outall = jax.lax.dot_general(xlT, p, (((1,), (0,)), ((), ())),
                                 preferred_element_type=jnp.float32)
    h_rows = [outall[h * C:(h + 1) * C, h, :] / den[h][None, :] for h in range(H)]
    hfullT = jnp.concatenate(h_rows, axis=0) + cb[:, None]

    inv_norm = jax.lax.rsqrt(jnp.sum(pw * pw))
    s = jnp.tanh(jax.lax.dot_general(pw[None, :], hfullT,
                                     (((1,), (0,)), ((), ())),
                                     preferred_element_type=jnp.float32) * inv_norm)
    sv = s[0]
    gt = (sv[:, None] > sv[None, :]).astype(jnp.float32)
    idx = jax.lax.broadcasted_iota(jnp.int32, (N, N), 0)
    idy = jax.lax.broadcasted_iota(jnp.int32, (N, N), 1)
    eq = ((sv[:, None] == sv[None, :]) & (idx < idy)).astype(jnp.float32)
    rank = jnp.sum(gt + eq, axis=0)
    w = jnp.where(rank < float(K), sv, 0.0)
    return jax.lax.dot_general(
        hfullT, w[:, None], (((1,), (0,)), ((), ())),
        preferred_element_type=jnp.float32)[:, 0] * (1.0 / K)


def _gat_topk_kernel(x_ref, adj_ref, wl_ref, bl_ref, wr_ref, br_ref,
                     att3_ref, we_ref, cb_ref, pw_ref, out_ref):
    wl = wl_ref[...]
    wr = wr_ref[...]
    att3 = att3_ref[...]
    bl, br, we, cb, pw = (bl_ref[0], br_ref[0], we_ref[0], cb_ref[0],
                          pw_ref[0])
    for g in range(PAIR):
        out_ref[g, 0, :] = _one_graph(x_ref[g], adj_ref[g], wl, bl, wr, br,
                                      att3, we, cb, pw)


@jax.jit
def kernel(x, adj, Wl, bl, Wr, br, att, We, conv_bias, pool_w):
    att3 = (jnp.eye(H, dtype=jnp.float32)[:, :, None] * att[None, :, :]).reshape(H, HC)
    out = pl.pallas_call(
        _gat_topk_kernel,
        grid=(B // PAIR,),
        in_specs=[
            pl.BlockSpec((PAIR, N, F), lambda b: (b, 0, 0)),
            pl.BlockSpec((PAIR, N, N), lambda b: (b, 0, 0)),
            pl.BlockSpec((F, HC), lambda b: (0, 0)),
            pl.BlockSpec((1, HC), lambda b: (0, 0)),
            pl.BlockSpec((F, HC), lambda b: (0, 0)),
            pl.BlockSpec((1, HC), lambda b: (0, 0)),
            pl.BlockSpec((H, HC), lambda b: (0, 0)),
            pl.BlockSpec((1, HC), lambda b: (0, 0)),
            pl.BlockSpec((1, HC), lambda b: (0, 0)),
            pl.BlockSpec((1, HC), lambda b: (0, 0)),
        ],
        out_specs=pl.BlockSpec((PAIR, 1, HC), lambda b: (b, 0, 0)),
        out_shape=jax.ShapeDtypeStruct((B, 1, HC), jnp.float32),
        compiler_params=pltpu.CompilerParams(
            dimension_semantics=("arbitrary",)),
    )(x, adj, Wl, bl.reshape(1, HC), Wr, br.reshape(1, HC), att3,
      We.reshape(1, HC), conv_bias.reshape(1, HC), pool_w.reshape(1, HC))
    return out[:, 0, :]


# speedup vs baseline: 1.0750x; 1.0506x over previous
import jax
import jax.numpy as jnp
import numpy as np
from jax.experimental import pallas as pl
from jax.experimental.pallas import tpu as pltpu

B, N, F = 8, 128, 256
H, C = 3, 32
HC = H * C
K = int(np.ceil(0.6 * N))
PAIR = 4


def _one_graph(xb, ab, wl, bl, wr, br, att3, we, cb, pw):
    xlT = jax.lax.dot_general(wl, xb, (((0,), (1,)), ((), ())),
                              preferred_element_type=jnp.float32) + bl[:, None]
    xrT = jax.lax.dot_general(wr, xb, (((0,), (1,)), ((), ())),
                              preferred_element_type=jnp.float32) + br[:, None]

    e3 = (xlT.T[:, :, None] + xrT[None, :, :]
          + ab[:, None, :] * we[None, :, None])
    e3 = jnp.maximum(e3, 0.2 * e3)
    att3b = jnp.broadcast_to(att3[None], (N, H, HC))
    logits = jax.lax.dot_general(att3b, e3, (((2,), (1,)), ((0,), (0,))),
                                 preferred_element_type=jnp.float32)

    mask3 = (ab != 0.0)[:, None, :]
    logits = jnp.where(mask3, logits, -1e9)
    m = jnp.max(logits, axis=0, keepdims=True)
    p = jnp.exp(logits - m)
    den = jnp.sum(p, axis=0)

    outall = jax.lax.dot_general(xlT, p, (((1,), (0,)), ((), ())),
                                 preferred_element_type=jnp.float32)
    h_rows = [outall[h * C:(h + 1) * C, h, :] / den[h][None, :] for h in range(H)]
    hfullT = jnp.concatenate(h_rows, axis=0) + cb[:, None]

    inv_norm = jax.lax.rsqrt(jnp.sum(pw * pw))
    s = jnp.tanh(jax.lax.dot_general(pw[None, :], hfullT,
                                     (((1,), (0,)), ((), ())),
                                     preferred_element_type=jnp.float32) * inv_norm)
    sv = s[0]
    gt = (sv[:, None] > sv[None, :]).astype(jnp.float32)
    idx = jax.lax.broadcasted_iota(jnp.int32, (N, N), 0)
    idy = jax.lax.broadcasted_iota(jnp.int32, (N, N), 1)
    eq = ((sv[:, None] == sv[None, :]) & (idx < idy)).astype(jnp.float32)
    rank = jnp.sum(gt + eq, axis=0)
    w = jnp.where(rank < float(K), sv, 0.0)
    return jax.lax.dot_general(
        hfullT, w[:, None], (((1,), (0,)), ((), ())),
        preferred_element_type=jnp.float32)[:, 0] * (1.0 / K)


def _gat_topk_kernel(x_ref, adj_ref, wl_ref, bl_ref, wr_ref, br_ref,
                     att3_ref, we_ref, cb_ref, pw_ref, out_ref):
    wl = wl_ref[...]
    wr = wr_ref[...]
    att = att3_ref[...]
    zc = jnp.zeros((1, C), dtype=jnp.float32)
    att3 = jnp.concatenate([
        jnp.concatenate([att[h][None, :] if g == h else zc for g in range(H)],
                        axis=1)
        for h in range(H)], axis=0)
    bl, br, we, cb, pw = (bl_ref[0], br_ref[0], we_ref[0], cb_ref[0],
                          pw_ref[0])
    for g in range(PAIR):
        out_ref[g, 0, :] = _one_graph(x_ref[g], adj_ref[g], wl, bl, wr, br,
                                      att3, we, cb, pw)


@jax.jit
def kernel(x, adj, Wl, bl, Wr, br, att, We, conv_bias, pool_w):
    out = pl.pallas_call(
        _gat_topk_kernel,
        grid=(B // PAIR,),
        in_specs=[
            pl.BlockSpec((PAIR, N, F), lambda b: (b, 0, 0)),
            pl.BlockSpec((PAIR, N, N), lambda b: (b, 0, 0)),
            pl.BlockSpec((F, HC), lambda b: (0, 0)),
            pl.BlockSpec((1, HC), lambda b: (0, 0)),
            pl.BlockSpec((F, HC), lambda b: (0, 0)),
            pl.BlockSpec((1, HC), lambda b: (0, 0)),
            pl.BlockSpec((H, C), lambda b: (0, 0)),
            pl.BlockSpec((1, HC), lambda b: (0, 0)),
            pl.BlockSpec((1, HC), lambda b: (0, 0)),
            pl.BlockSpec((1, HC), lambda b: (0, 0)),
        ],
        out_specs=pl.BlockSpec((PAIR, 1, HC), lambda b: (b, 0, 0)),
        out_shape=jax.ShapeDtypeStruct((B, 1, HC), jnp.float32),
        compiler_params=pltpu.CompilerParams(
            dimension_semantics=("arbitrary",)),
    )(x, adj, Wl, bl.reshape(1, HC), Wr, br.reshape(1, HC), att,
      We.reshape(1, HC), conv_bias.reshape(1, HC), pool_w.reshape(1, HC))
    return out[:, 0, :]
